# baseline (speedup 1.0000x reference)
import functools

import jax
import jax.numpy as jnp
import numpy as _np
from jax import lax
from jax.experimental import pallas as pl
from jax.experimental.pallas import tpu as pltpu
from jax.experimental.pallas import tpu_sc as plsc

B, S, F, V = 16, 256, 4096, 30
L = 16
N = B * V
NPAD = 512
NSC = 16
EPT = NPAD // NSC
EV = EPT // L

_BIDX = _np.minimum(_np.arange(NPAD) // V, B - 1).astype(_np.int32)


def _body(extra, packed, vlen, out,
          ent_v, win_v, pad_v, shared, sum_v, vlen_v, out_v, sem):
    cid = lax.axis_index("c")
    sid = lax.axis_index("s")

    @pl.when(cid == 0)
    def _():
        d0 = pltpu.async_copy(packed.at[pl.ds(sid * 3 * EPT, 3 * EPT)],
                              ent_v, sem)
        dv = pltpu.async_copy(vlen, vlen_v, sem)
        d0.wait()
        dv.wait()

        descs = []
        for i in range(EV):
            bv = ent_v[pl.ds(2 * EPT + i * L, L)]
            sv = (ent_v[pl.ds(i * L, L)] >> 3) << 3
            fv = (ent_v[pl.ds(EPT + i * L, L)] >> 7) << 7
            for k in range(L):
                descs.append(pltpu.async_copy(
                    extra.at[bv[k],
                             pl.ds(pl.multiple_of(sv[k], 8), 8),
                             pl.ds(pl.multiple_of(fv[k], 128), 128)],
                    win_v.at[pl.ds((i * L + k) * 8, 8)], sem))
        for d in descs:
            d.wait()

        acc = jnp.zeros((L,), jnp.float32)
        ii = lax.iota(jnp.int32, L)
        for i in range(EV):
            vlv = ent_v[pl.ds(i * L, L)]
            fv = ent_v[pl.ds(EPT + i * L, L)]
            subl = vlv & 7
            col0 = (fv & 127) & ~(L - 1)
            lane = jnp.where(vlv != 0, fv & (L - 1), L)
            for k in range(L):
                row = win_v[(i * L + k) * 8 + subl[k], pl.ds(col0[k], L)]
                acc = acc + jnp.where(ii == lane[k], row, 0.0)

        pad_v[0, pl.ds(0, L)] = acc
        pltpu.sync_copy(pad_v, shared.at[sid])
        plsc.subcore_barrier()

        @pl.when(sid == 0)
        def _():
            pltpu.sync_copy(shared, sum_v)
            tot = sum_v[0, 0, pl.ds(0, L)]
            for t in range(1, NSC):
                tot = tot + sum_v[t, 0, pl.ds(0, L)]
            nv = vlen_v[...]
            total = tot[0]
            norm = nv[0]
            for i in range(1, L):
                total = total + tot[i]
                norm = norm + nv[i]
            tvec = jnp.full((L,), total, jnp.float32)
            nvec = jnp.full((L,), norm, jnp.float32)
            out_v[...] = -tvec / nvec
            pltpu.sync_copy(out_v, out)


@functools.cache
def _get_call():
    return pl.kernel(
        _body,
        out_type=jax.ShapeDtypeStruct((L,), jnp.float32),
        mesh=plsc.VectorSubcoreMesh(core_axis_name="c", subcore_axis_name="s",
                                    num_cores=1),
        compiler_params=pltpu.CompilerParams(
            use_tc_tiling_on_sc=True,
            skip_device_barrier=True,
            disable_bounds_checks=True,
            disable_semaphore_checks=True,
        ),
        scratch_types=[
            pltpu.VMEM((3 * EPT,), jnp.int32),
            pltpu.VMEM((EPT * 8, 128), jnp.float32),
            pltpu.VMEM((8, 128), jnp.float32),
            pltpu.VMEM_SHARED((NSC, 8, 128), jnp.float32),
            pltpu.VMEM((NSC, 8, 128), jnp.float32),
            pltpu.VMEM((L,), jnp.int32),
            pltpu.VMEM((L,), jnp.float32),
            pltpu.SemaphoreType.DMA,
        ],
    )


def kernel(log_pa, score, v_label, v_l, role_label, roleset_id, extra):
    vl = jnp.zeros((NPAD,), jnp.int32).at[:N].set(
        v_label.reshape(-1).astype(jnp.int32))
    rs = jnp.zeros((NPAD,), jnp.int32).at[:N].set(
        roleset_id.reshape(-1).astype(jnp.int32))
    packed = jnp.concatenate(
        [vl.reshape(NSC, EPT), rs.reshape(NSC, EPT),
         jnp.asarray(_BIDX).reshape(NSC, EPT)], axis=1).reshape(-1)
    out = _get_call()(extra, packed, v_l.astype(jnp.int32))
    return out[0]


# speedup vs baseline: 1.0562x; 1.0562x over previous
import functools

import jax
import jax.numpy as jnp
import numpy as _np
from jax import lax
from jax.experimental import pallas as pl
from jax.experimental.pallas import tpu as pltpu
from jax.experimental.pallas import tpu_sc as plsc

B, S, F, V = 16, 256, 4096, 30
L = 16
N = B * V
NPAD = 512
NSC = 16
EPT = NPAD // NSC
EV = EPT // L

_BIDX = _np.minimum(_np.arange(NPAD) // V, B - 1).astype(_np.int32)


def _body(extra, packed, vlen, out,
          ent_v, win_v, pad_v, shared, sum_v, vlen_v, out_v, sem):
    cid = lax.axis_index("c")
    sid = lax.axis_index("s")

    @pl.when(cid == 0)
    def _():
        d0 = pltpu.async_copy(packed.at[pl.ds(sid * 3 * EPT, 3 * EPT)],
                              ent_v, sem)
        dv = pltpu.async_copy(vlen, vlen_v, sem)
        d0.wait()
        dv.wait()

        descs = []
        for i in range(EV):
            bv = ent_v[pl.ds(2 * EPT + i * L, L)]
            sv = (ent_v[pl.ds(i * L, L)] >> 3) << 3
            fv = (ent_v[pl.ds(EPT + i * L, L)] >> 7) << 7
            for k in range(L):
                descs.append(pltpu.async_copy(
                    extra.at[bv[k],
                             pl.ds(pl.multiple_of(sv[k], 8), 8),
                             pl.ds(pl.multiple_of(fv[k], 128), 128)],
                    win_v.at[pl.ds((i * L + k) * 8, 8)], sem))
        acc = jnp.zeros((L,), jnp.float32)
        ii = lax.iota(jnp.int32, L)
        for i in range(EV):
            vlv = ent_v[pl.ds(i * L, L)]
            fv = ent_v[pl.ds(EPT + i * L, L)]
            subl = vlv & 7
            col0 = (fv & 127) & ~(L - 1)
            lane = jnp.where(vlv != 0, fv & (L - 1), L)
            for k in range(L):
                descs[i * L + k].wait()
                row = win_v[(i * L + k) * 8 + subl[k], pl.ds(col0[k], L)]
                acc = acc + jnp.where(ii == lane[k], row, 0.0)

        pad_v[0, pl.ds(0, L)] = acc
        pltpu.sync_copy(pad_v, shared.at[sid])
        plsc.subcore_barrier()

        @pl.when(sid == 0)
        def _():
            pltpu.sync_copy(shared.at[:, pl.ds(0, 1), :], sum_v)
            tot = sum_v[0, 0, pl.ds(0, L)]
            for t in range(1, NSC):
                tot = tot + sum_v[t, 0, pl.ds(0, L)]
            nv = vlen_v[...]
            total = tot[0]
            norm = nv[0]
            for i in range(1, L):
                total = total + tot[i]
                norm = norm + nv[i]
            tvec = jnp.full((L,), total, jnp.float32)
            nvec = jnp.full((L,), norm, jnp.float32)
            out_v[...] = -tvec / nvec
            pltpu.sync_copy(out_v, out)


@functools.cache
def _get_call():
    return pl.kernel(
        _body,
        out_type=jax.ShapeDtypeStruct((L,), jnp.float32),
        mesh=plsc.VectorSubcoreMesh(core_axis_name="c", subcore_axis_name="s",
                                    num_cores=1),
        compiler_params=pltpu.CompilerParams(use_tc_tiling_on_sc=True),
        scratch_types=[
            pltpu.VMEM((3 * EPT,), jnp.int32),
            pltpu.VMEM((EPT * 8, 128), jnp.float32),
            pltpu.VMEM((8, 128), jnp.float32),
            pltpu.VMEM_SHARED((NSC, 8, 128), jnp.float32),
            pltpu.VMEM((NSC, 1, 128), jnp.float32),
            pltpu.VMEM((L,), jnp.int32),
            pltpu.VMEM((L,), jnp.float32),
            pltpu.SemaphoreType.DMA,
        ],
    )


def kernel(log_pa, score, v_label, v_l, role_label, roleset_id, extra):
    vl = jnp.zeros((NPAD,), jnp.int32).at[:N].set(
        v_label.reshape(-1).astype(jnp.int32))
    rs = jnp.zeros((NPAD,), jnp.int32).at[:N].set(
        roleset_id.reshape(-1).astype(jnp.int32))
    packed = jnp.concatenate(
        [vl.reshape(NSC, EPT), rs.reshape(NSC, EPT),
         jnp.asarray(_BIDX).reshape(NSC, EPT)], axis=1).reshape(-1)
    out = _get_call()(extra, packed, v_l.astype(jnp.int32))
    return out[0]


# speedup vs baseline: 1.0848x; 1.0270x over previous
import functools

import jax
import jax.numpy as jnp
from jax import lax
from jax.experimental import pallas as pl
from jax.experimental.pallas import tpu as pltpu
from jax.experimental.pallas import tpu_sc as plsc

B, S, F, V = 16, 256, 4096, 30
L = 16
NSC = 16
EV = 2
HI = V - L


def _body(extra, vlab, rs, vlen, out,
          vlab_v, rs_v, win_v, pad_v, shared, sum_v, vlen_v, out_v, sem):
    sid = lax.axis_index("s")

    d0 = pltpu.async_copy(vlab, vlab_v, sem)
    d1 = pltpu.async_copy(rs, rs_v, sem)
    d2 = pltpu.async_copy(vlen, vlen_v, sem)
    d0.wait()
    d1.wait()
    d2.wait()

    starts = (0, HI)
    descs = []
    for i in range(EV):
        sv = (vlab_v[sid, pl.ds(starts[i], L)] >> 3) << 3
        fv = (rs_v[sid, pl.ds(starts[i], L)] >> 7) << 7
        for k in range(L):
            descs.append(pltpu.async_copy(
                extra.at[sid,
                         pl.ds(pl.multiple_of(sv[k], 8), 8),
                         pl.ds(pl.multiple_of(fv[k], 128), 128)],
                win_v.at[pl.ds((i * L + k) * 8, 8)], sem))

    acc = jnp.zeros((L,), jnp.float32)
    ii = lax.iota(jnp.int32, L)
    for i in range(EV):
        vlv = vlab_v[sid, pl.ds(starts[i], L)]
        fv = rs_v[sid, pl.ds(starts[i], L)]
        subl = vlv & 7
        col0 = (fv & 127) & ~(L - 1)
        live = vlv != 0
        if i == 1:
            live = live & (ii >= 2 * L - V)
        lane = jnp.where(live, fv & (L - 1), L)
        for k in range(L):
            descs[i * L + k].wait()
            row = win_v[(i * L + k) * 8 + subl[k], pl.ds(col0[k], L)]
            acc = acc + jnp.where(ii == lane[k], row, 0.0)

    pad_v[0, pl.ds(0, L)] = acc
    pltpu.sync_copy(pad_v, shared.at[sid])
    plsc.subcore_barrier()

    @pl.when(sid == 0)
    def _():
        pltpu.sync_copy(shared.at[:, pl.ds(0, 1), :], sum_v)
        tot = sum_v[0, 0, pl.ds(0, L)]
        for t in range(1, NSC):
            tot = tot + sum_v[t, 0, pl.ds(0, L)]
        nv = vlen_v[...]
        total = tot[0]
        norm = nv[0]
        for i in range(1, L):
            total = total + tot[i]
            norm = norm + nv[i]
        tvec = jnp.full((L,), total, jnp.float32)
        nvec = jnp.full((L,), norm, jnp.float32)
        out_v[...] = -tvec / nvec
        pltpu.sync_copy(out_v, out)


@functools.cache
def _get_call():
    return pl.kernel(
        _body,
        out_type=jax.ShapeDtypeStruct((L,), jnp.float32),
        mesh=plsc.VectorSubcoreMesh(core_axis_name="c", subcore_axis_name="s",
                                    num_cores=1),
        compiler_params=pltpu.CompilerParams(use_tc_tiling_on_sc=True),
        scratch_types=[
            pltpu.VMEM((B, V), jnp.int32),
            pltpu.VMEM((B, V), jnp.int32),
            pltpu.VMEM((EV * L * 8, 128), jnp.float32),
            pltpu.VMEM((8, 128), jnp.float32),
            pltpu.VMEM_SHARED((NSC, 8, 128), jnp.float32),
            pltpu.VMEM((NSC, 1, 128), jnp.float32),
            pltpu.VMEM((L,), jnp.int32),
            pltpu.VMEM((L,), jnp.float32),
            pltpu.SemaphoreType.DMA,
        ],
    )


def kernel(log_pa, score, v_label, v_l, role_label, roleset_id, extra):
    out = _get_call()(extra, v_label.astype(jnp.int32),
                      roleset_id.astype(jnp.int32), v_l.astype(jnp.int32))
    return out[0]
